# baseline (speedup 1.0000x reference)
import functools
import jax
import jax.numpy as jnp
from jax import lax
from jax.experimental import pallas as pl
from jax.experimental.pallas import tpu as pltpu
from jax.experimental.pallas import tpu_sc as plsc

N_NODES = 10000
N_PAD = 10240
E_TOTAL = 320000
NC, NS = 2, 16
NW = NC * NS
L = 16


def _pack_body(e_ref, p_ref):
    p_ref[...] = e_ref[0:1, :] * 32768 + e_ref[1:2, :]


def _pack_edges(edge_index):
    E = edge_index.shape[1]
    blk = 2560
    return pl.pallas_call(
        _pack_body,
        grid=(E // blk,),
        in_specs=[pl.BlockSpec((2, blk), lambda i: (0, i))],
        out_specs=pl.BlockSpec((1, blk), lambda i: (0, i)),
        out_shape=jax.ShapeDtypeStruct((1, E), jnp.int32),
    )(edge_index)


def _proj1_body(xT_ref, wT_ref, alT_ref, arT_ref, fT_ref, elT_ref, erT_ref):
    fT = jnp.dot(wT_ref[...], xT_ref[...], preferred_element_type=jnp.float32)
    fT_ref[...] = fT
    elT_ref[...] = jnp.dot(alT_ref[...], fT, preferred_element_type=jnp.float32)
    erT_ref[...] = jnp.dot(arT_ref[...], fT, preferred_element_type=jnp.float32)


def _tc_proj1(xT, W1T, AL1T, AR1T):
    blk = 512
    g = N_PAD // blk
    return pl.pallas_call(
        _proj1_body,
        grid=(g,),
        in_specs=[
            pl.BlockSpec((128, blk), lambda i: (0, i)),
            pl.BlockSpec((256, 128), lambda i: (0, 0)),
            pl.BlockSpec((4, 256), lambda i: (0, 0)),
            pl.BlockSpec((4, 256), lambda i: (0, 0)),
        ],
        out_specs=[
            pl.BlockSpec((256, blk), lambda i: (0, i)),
            pl.BlockSpec((4, blk), lambda i: (0, i)),
            pl.BlockSpec((4, blk), lambda i: (0, i)),
        ],
        out_shape=[
            jax.ShapeDtypeStruct((256, N_PAD), jnp.float32),
            jax.ShapeDtypeStruct((4, N_PAD), jnp.float32),
            jax.ShapeDtypeStruct((4, N_PAD), jnp.float32),
        ],
    )(xT, W1T, AL1T, AR1T)


def _mid_body(o1T_ref, dP_ref, w2T_ref, al2T_ref, ar2T_ref, b1_ref,
              f2T_ref, el2T_ref, er2T_ref):
    den = jnp.sum(dP_ref[...].reshape(4, 8, o1T_ref.shape[1]), axis=1)
    x = o1T_ref[...].reshape(4, 64, o1T_ref.shape[1]) / (den[:, None, :] + 1e-16)
    x = x.reshape(256, o1T_ref.shape[1]) + b1_ref[...]
    h1T = jnp.where(x > 0, x, jnp.exp(x) - 1.0)
    f2T = jnp.dot(w2T_ref[...], h1T, preferred_element_type=jnp.float32)
    f2T_ref[...] = f2T
    el2T_ref[...] = jnp.dot(al2T_ref[...], f2T, preferred_element_type=jnp.float32)
    er2T_ref[...] = jnp.dot(ar2T_ref[...], f2T, preferred_element_type=jnp.float32)


def _tc_mid(out1T, denP1, W2T, AL2T, AR2T, b1col):
    blk = 512
    g = N_PAD // blk
    return pl.pallas_call(
        _mid_body,
        grid=(g,),
        in_specs=[
            pl.BlockSpec((256, blk), lambda i: (0, i)),
            pl.BlockSpec((32, blk), lambda i: (0, i)),
            pl.BlockSpec((48, 256), lambda i: (0, 0)),
            pl.BlockSpec((1, 48), lambda i: (0, 0)),
            pl.BlockSpec((1, 48), lambda i: (0, 0)),
            pl.BlockSpec((256, 1), lambda i: (0, 0)),
        ],
        out_specs=[
            pl.BlockSpec((48, blk), lambda i: (0, i)),
            pl.BlockSpec((1, blk), lambda i: (0, i)),
            pl.BlockSpec((1, blk), lambda i: (0, i)),
        ],
        out_shape=[
            jax.ShapeDtypeStruct((48, N_PAD), jnp.float32),
            jax.ShapeDtypeStruct((1, N_PAD), jnp.float32),
            jax.ShapeDtypeStruct((1, N_PAD), jnp.float32),
        ],
    )(out1T, denP1, W2T, AL2T, AR2T, b1col)


def _final_body(o2T_ref, dP_ref, b2_ref, out_ref):
    den = jnp.sum(dP_ref[...], axis=0, keepdims=True)
    out_ref[...] = o2T_ref[...] / (den + 1e-16) + b2_ref[...]


def _tc_final(out2T, denP2, b2col):
    blk = 512
    g = N_PAD // blk
    return pl.pallas_call(
        _final_body,
        grid=(g,),
        in_specs=[
            pl.BlockSpec((48, blk), lambda i: (0, i)),
            pl.BlockSpec((32, blk), lambda i: (0, i)),
            pl.BlockSpec((48, 1), lambda i: (0, 0)),
        ],
        out_specs=pl.BlockSpec((48, blk), lambda i: (0, i)),
        out_shape=jax.ShapeDtypeStruct((48, N_PAD), jnp.float32),
    )(out2T, denP2, b2col)


@functools.lru_cache(maxsize=None)
def _mesh():
    return plsc.VectorSubcoreMesh(
        core_axis_name="c", subcore_axis_name="s",
        num_cores=NC, num_subcores=NS)


def _worker_id():
    return lax.axis_index("s") * NC + lax.axis_index("c")


def _zero_vmem(ref, n):
    z = jnp.zeros((L,), jnp.float32)

    def zb(i, _):
        ref[pl.ds(i * L, L)] = z
        return 0

    lax.fori_loop(0, n // L, zb, 0)


def _make_sc_stage1(H, NJ, CH):
    EPW = E_TOTAL // NJ

    @functools.partial(
        pl.kernel,
        mesh=_mesh(),
        compiler_params=pltpu.CompilerParams(needs_layout_passes=False),
        out_type=[
            jax.ShapeDtypeStruct((H * E_TOTAL,), jnp.float32),
            jax.ShapeDtypeStruct((H * NJ, N_PAD), jnp.float32),
        ],
        scratch_types=[
            pltpu.VMEM((N_PAD,), jnp.float32),
            pltpu.VMEM((N_PAD,), jnp.float32),
            pltpu.VMEM((N_PAD,), jnp.float32),
            pltpu.VMEM((CH,), jnp.int32),
            pltpu.VMEM((CH,), jnp.float32),
        ],
    )
    def stage1(packed, elT, erT, ee_out, denP, el_v, er_v, den_v, pk_v, ee_v):
        wid = _worker_id()
        h = wid // NJ
        j = wid % NJ
        pltpu.sync_copy(elT.at[h], el_v)
        pltpu.sync_copy(erT.at[h], er_v)
        _zero_vmem(den_v, N_PAD)
        base = j * EPW

        def chunk_body(ci, _):
            off = base + ci * CH
            pltpu.sync_copy(packed.at[pl.ds(off, CH)], pk_v)

            @plsc.parallel_loop(0, CH // L, 1, unroll=8)
            def ib(k):
                pk = pk_v[pl.ds(k * L, L)]
                s = pk >> 15
                d = pk & 32767
                e = plsc.load_gather(el_v, [s]) + plsc.load_gather(er_v, [d])
                e = jnp.where(e > 0, e, 0.2 * e)
                ex = jnp.exp(e)
                ee_v[pl.ds(k * L, L)] = ex
                plsc.addupdate_scatter(den_v, [d], ex)
            pltpu.sync_copy(ee_v, ee_out.at[pl.ds(h * E_TOTAL + off, CH)])
            return 0

        lax.fori_loop(0, EPW // CH, chunk_body, 0)
        pltpu.sync_copy(den_v, denP.at[h * NJ + j])

    return stage1


def _make_sc_stage2(C, CPW, NCHUNK, CPH, CH):
    NCH = E_TOTAL // CH
    assert NCH % 2 == 0

    @functools.partial(
        pl.kernel,
        mesh=_mesh(),
        compiler_params=pltpu.CompilerParams(needs_layout_passes=False),
        out_type=jax.ShapeDtypeStruct((C, N_PAD), jnp.float32),
        scratch_types=[
            pltpu.VMEM((CPW * N_PAD,), jnp.float32),
            pltpu.VMEM((CPW * N_PAD,), jnp.float32),
            pltpu.VMEM((CH,), jnp.int32),
            pltpu.VMEM((CH,), jnp.int32),
            pltpu.VMEM((CH,), jnp.float32),
            pltpu.VMEM((CH,), jnp.float32),
            pltpu.SemaphoreType.DMA,
            pltpu.SemaphoreType.DMA,
            pltpu.SemaphoreType.DMA,
            pltpu.SemaphoreType.DMA,
        ],
    )
    def stage2(packed, featT, ee_hbm, outT, tbl, acc,
               pk0, pk1, ee0, ee1, sp0, sp1, se0, se1):
        wid = _worker_id()
        pk_bufs = (pk0, pk1)
        ee_bufs = (ee0, ee1)
        sp_sems = (sp0, sp1)
        se_sems = (se0, se1)

        def do_chunk(chunk):
            c0 = chunk * CPW
            h = chunk // CPH
            ee_base = h * E_TOTAL
            for c in range(CPW):
                pltpu.sync_copy(featT.at[c0 + c], tbl.at[pl.ds(c * N_PAD, N_PAD)])
            _zero_vmem(acc, CPW * N_PAD)

            pltpu.make_async_copy(packed.at[pl.ds(0, CH)], pk0, sp0).start()
            pltpu.make_async_copy(ee_hbm.at[pl.ds(ee_base, CH)], ee0, se0).start()

            def pair_body(cp, _):
                for b in range(2):
                    ci = cp * 2 + b
                    pk_v, ee_v = pk_bufs[b], ee_bufs[b]
                    npk, nee = pk_bufs[1 - b], ee_bufs[1 - b]
                    pltpu.make_async_copy(
                        packed.at[pl.ds(ci * CH, CH)], pk_v, sp_sems[b]).wait()
                    pltpu.make_async_copy(
                        ee_hbm.at[pl.ds(ee_base + ci * CH, CH)], ee_v,
                        se_sems[b]).wait()

                    @pl.when(ci + 1 < NCH)
                    def _():
                        off2 = (ci + 1) * CH
                        pltpu.make_async_copy(
                            packed.at[pl.ds(off2, CH)], npk,
                            sp_sems[1 - b]).start()
                        pltpu.make_async_copy(
                            ee_hbm.at[pl.ds(ee_base + off2, CH)], nee,
                            se_sems[1 - b]).start()

                    @plsc.parallel_loop(0, CH // L, 1, unroll=16)
                    def ib(k):
                        pk = pk_v[pl.ds(k * L, L)]
                        s = pk >> 15
                        d = pk & 32767
                        ex = ee_v[pl.ds(k * L, L)]
                        for c in range(CPW):
                            g = plsc.load_gather(tbl, [s + (c * N_PAD)])
                            plsc.addupdate_scatter(acc, [d + (c * N_PAD)], g * ex)

                return 0

            lax.fori_loop(0, NCH // 2, pair_body, 0)
            for c in range(CPW):
                pltpu.sync_copy(acc.at[pl.ds(c * N_PAD, N_PAD)], outT.at[c0 + c])

        npass = (NCHUNK + NW - 1) // NW
        for p in range(npass):
            if (p + 1) * NW <= NCHUNK:
                do_chunk(wid + p * NW)
            else:
                @pl.when(wid < NCHUNK - p * NW)
                def _():
                    do_chunk(wid + p * NW)

    return stage2


_make_sc_stage1 = functools.lru_cache(maxsize=None)(_make_sc_stage1)
_make_sc_stage2 = functools.lru_cache(maxsize=None)(_make_sc_stage2)


def _head_proj_mat(a):
    H, D = a.shape
    return (jnp.eye(H, dtype=a.dtype)[:, :, None] * a[None, :, :]).reshape(H, H * D)


def kernel(edge_index, x, W1, al1, ar1, b1, W2, al2, ar2, b2):
    packed = _pack_edges(edge_index).reshape(E_TOTAL)

    xT = jnp.pad(x.T, ((0, 0), (0, N_PAD - N_NODES)))
    W1T = W1.T
    AL1T = _head_proj_mat(al1)
    AR1T = _head_proj_mat(ar1)
    feat1T, el1T, er1T = _tc_proj1(xT, W1T, AL1T, AR1T)

    ee1, denP1 = _make_sc_stage1(H=4, NJ=8, CH=4000)(packed, el1T, er1T)
    out1T = _make_sc_stage2(C=256, CPW=4, NCHUNK=64, CPH=16, CH=8000)(
        packed, feat1T, ee1)

    W2T = jnp.pad(W2.T, ((0, 1), (0, 0)))
    AL2T = jnp.pad(al2, ((0, 0), (0, 1)))
    AR2T = jnp.pad(ar2, ((0, 0), (0, 1)))
    b1col = b1[:, None]
    feat2T, el2T, er2T = _tc_mid(out1T, denP1, W2T, AL2T, AR2T, b1col)

    ee2, denP2 = _make_sc_stage1(H=1, NJ=32, CH=2000)(packed, el2T, er2T)
    out2T = _make_sc_stage2(C=48, CPW=3, NCHUNK=16, CPH=16, CH=8000)(
        packed, feat2T, ee2)

    b2col = jnp.pad(b2, (0, 1))[:, None]
    res = _tc_final(out2T, denP2, b2col)
    return res[:47, :N_NODES].T


# speedup vs baseline: 1.3248x; 1.3248x over previous
import functools
import jax
import jax.numpy as jnp
from jax import lax
from jax.experimental import pallas as pl
from jax.experimental.pallas import tpu as pltpu
from jax.experimental.pallas import tpu_sc as plsc

N_NODES = 10000
N_PAD = 10240
E_TOTAL = 320000
NC, NS = 2, 16
NW = NC * NS
L = 16


def _pack_body(e_ref, p_ref):
    p_ref[...] = e_ref[0:1, :] * 32768 + e_ref[1:2, :]


def _pack_edges(edge_index):
    E = edge_index.shape[1]
    blk = 2560
    return pl.pallas_call(
        _pack_body,
        grid=(E // blk,),
        in_specs=[pl.BlockSpec((2, blk), lambda i: (0, i))],
        out_specs=pl.BlockSpec((1, blk), lambda i: (0, i)),
        out_shape=jax.ShapeDtypeStruct((1, E), jnp.int32),
    )(edge_index)


def _proj1_body(xT_ref, wT_ref, alT_ref, arT_ref, fT_ref, elT_ref, erT_ref):
    fT = jnp.dot(wT_ref[...], xT_ref[...], preferred_element_type=jnp.float32)
    fT_ref[...] = fT
    elT_ref[...] = jnp.dot(alT_ref[...], fT, preferred_element_type=jnp.float32)
    erT_ref[...] = jnp.dot(arT_ref[...], fT, preferred_element_type=jnp.float32)


def _tc_proj1(xT, W1T, AL1T, AR1T):
    blk = 512
    g = N_PAD // blk
    return pl.pallas_call(
        _proj1_body,
        grid=(g,),
        in_specs=[
            pl.BlockSpec((128, blk), lambda i: (0, i)),
            pl.BlockSpec((256, 128), lambda i: (0, 0)),
            pl.BlockSpec((4, 256), lambda i: (0, 0)),
            pl.BlockSpec((4, 256), lambda i: (0, 0)),
        ],
        out_specs=[
            pl.BlockSpec((256, blk), lambda i: (0, i)),
            pl.BlockSpec((4, blk), lambda i: (0, i)),
            pl.BlockSpec((4, blk), lambda i: (0, i)),
        ],
        out_shape=[
            jax.ShapeDtypeStruct((256, N_PAD), jnp.float32),
            jax.ShapeDtypeStruct((4, N_PAD), jnp.float32),
            jax.ShapeDtypeStruct((4, N_PAD), jnp.float32),
        ],
    )(xT, W1T, AL1T, AR1T)


def _mid_body(o1T_ref, dP_ref, w2T_ref, al2T_ref, ar2T_ref, b1_ref,
              f2T_ref, el2T_ref, er2T_ref):
    den = jnp.sum(dP_ref[...].reshape(4, 8, o1T_ref.shape[1]), axis=1)
    x = o1T_ref[...].reshape(4, 64, o1T_ref.shape[1]) / (den[:, None, :] + 1e-16)
    x = x.reshape(256, o1T_ref.shape[1]) + b1_ref[...]
    h1T = jnp.where(x > 0, x, jnp.exp(x) - 1.0)
    f2T = jnp.dot(w2T_ref[...], h1T, preferred_element_type=jnp.float32)
    f2T_ref[...] = f2T
    el2T_ref[...] = jnp.dot(al2T_ref[...], f2T, preferred_element_type=jnp.float32)
    er2T_ref[...] = jnp.dot(ar2T_ref[...], f2T, preferred_element_type=jnp.float32)


def _tc_mid(out1T, denP1, W2T, AL2T, AR2T, b1col):
    blk = 512
    g = N_PAD // blk
    return pl.pallas_call(
        _mid_body,
        grid=(g,),
        in_specs=[
            pl.BlockSpec((256, blk), lambda i: (0, i)),
            pl.BlockSpec((32, blk), lambda i: (0, i)),
            pl.BlockSpec((48, 256), lambda i: (0, 0)),
            pl.BlockSpec((1, 48), lambda i: (0, 0)),
            pl.BlockSpec((1, 48), lambda i: (0, 0)),
            pl.BlockSpec((256, 1), lambda i: (0, 0)),
        ],
        out_specs=[
            pl.BlockSpec((48, blk), lambda i: (0, i)),
            pl.BlockSpec((1, blk), lambda i: (0, i)),
            pl.BlockSpec((1, blk), lambda i: (0, i)),
        ],
        out_shape=[
            jax.ShapeDtypeStruct((48, N_PAD), jnp.float32),
            jax.ShapeDtypeStruct((1, N_PAD), jnp.float32),
            jax.ShapeDtypeStruct((1, N_PAD), jnp.float32),
        ],
    )(out1T, denP1, W2T, AL2T, AR2T, b1col)


def _final_body(o2T_ref, dP_ref, b2_ref, out_ref):
    den = jnp.sum(dP_ref[...], axis=0, keepdims=True)
    out_ref[...] = o2T_ref[...] / (den + 1e-16) + b2_ref[...]


def _tc_final(out2T, denP2, b2col):
    blk = 512
    g = N_PAD // blk
    return pl.pallas_call(
        _final_body,
        grid=(g,),
        in_specs=[
            pl.BlockSpec((48, blk), lambda i: (0, i)),
            pl.BlockSpec((32, blk), lambda i: (0, i)),
            pl.BlockSpec((48, 1), lambda i: (0, 0)),
        ],
        out_specs=pl.BlockSpec((48, blk), lambda i: (0, i)),
        out_shape=jax.ShapeDtypeStruct((48, N_PAD), jnp.float32),
    )(out2T, denP2, b2col)


@functools.lru_cache(maxsize=None)
def _mesh():
    return plsc.VectorSubcoreMesh(
        core_axis_name="c", subcore_axis_name="s",
        num_cores=NC, num_subcores=NS)


def _worker_id():
    return lax.axis_index("s") * NC + lax.axis_index("c")


def _zero_vmem(ref, n):
    z = jnp.zeros((L,), jnp.float32)

    def zb(i, _):
        ref[pl.ds(i * L, L)] = z
        return 0

    lax.fori_loop(0, n // L, zb, 0)


def _make_sc_stage1(H, NJ, CH):
    EPW = E_TOTAL // NJ

    @functools.partial(
        pl.kernel,
        mesh=_mesh(),
        compiler_params=pltpu.CompilerParams(needs_layout_passes=False),
        out_type=[
            jax.ShapeDtypeStruct((H * E_TOTAL,), jnp.float32),
            jax.ShapeDtypeStruct((H * NJ, N_PAD), jnp.float32),
        ],
        scratch_types=[
            pltpu.VMEM((N_PAD,), jnp.float32),
            pltpu.VMEM((N_PAD,), jnp.float32),
            pltpu.VMEM((N_PAD,), jnp.float32),
            pltpu.VMEM((CH,), jnp.int32),
            pltpu.VMEM((CH,), jnp.float32),
        ],
    )
    def stage1(packed, elT, erT, ee_out, denP, el_v, er_v, den_v, pk_v, ee_v):
        wid = _worker_id()
        h = wid // NJ
        j = wid % NJ
        pltpu.sync_copy(elT.at[h], el_v)
        pltpu.sync_copy(erT.at[h], er_v)
        _zero_vmem(den_v, N_PAD)
        base = j * EPW

        def chunk_body(ci, _):
            off = base + ci * CH
            pltpu.sync_copy(packed.at[pl.ds(off, CH)], pk_v)

            @plsc.parallel_loop(0, CH // L, 1, unroll=8)
            def ib(k):
                pk = pk_v[pl.ds(k * L, L)]
                s = pk >> 15
                d = pk & 32767
                e = plsc.load_gather(el_v, [s]) + plsc.load_gather(er_v, [d])
                e = jnp.where(e > 0, e, 0.2 * e)
                ex = jnp.exp(e)
                ee_v[pl.ds(k * L, L)] = ex
                plsc.addupdate_scatter(den_v, [d], ex)
            pltpu.sync_copy(ee_v, ee_out.at[pl.ds(h * E_TOTAL + off, CH)])
            return 0

        lax.fori_loop(0, EPW // CH, chunk_body, 0)
        pltpu.sync_copy(den_v, denP.at[h * NJ + j])

    return stage1


def _make_sc_stage2(C, CPW, NCHUNK, CPH, CH):
    NCH = E_TOTAL // CH
    assert NCH % 2 == 0

    @functools.partial(
        pl.kernel,
        mesh=_mesh(),
        compiler_params=pltpu.CompilerParams(needs_layout_passes=False),
        out_type=jax.ShapeDtypeStruct((C, N_PAD), jnp.float32),
        scratch_types=[
            pltpu.VMEM((CPW * N_PAD,), jnp.float32),
            pltpu.VMEM((CPW * N_PAD,), jnp.float32),
            pltpu.VMEM((CH,), jnp.int32),
            pltpu.VMEM((CH,), jnp.int32),
            pltpu.VMEM((CH,), jnp.float32),
            pltpu.VMEM((CH,), jnp.float32),
            pltpu.SemaphoreType.DMA,
            pltpu.SemaphoreType.DMA,
            pltpu.SemaphoreType.DMA,
            pltpu.SemaphoreType.DMA,
        ],
    )
    def stage2(packed, featT, ee_hbm, outT, tbl, acc,
               pk0, pk1, ee0, ee1, sp0, sp1, se0, se1):
        wid = _worker_id()
        pk_bufs = (pk0, pk1)
        ee_bufs = (ee0, ee1)
        sp_sems = (sp0, sp1)
        se_sems = (se0, se1)

        def do_chunk(chunk):
            c0 = chunk * CPW
            h = chunk // CPH
            ee_base = h * E_TOTAL
            for c in range(CPW):
                pltpu.sync_copy(featT.at[c0 + c], tbl.at[pl.ds(c * N_PAD, N_PAD)])
            _zero_vmem(acc, CPW * N_PAD)

            pltpu.make_async_copy(packed.at[pl.ds(0, CH)], pk0, sp0).start()
            pltpu.make_async_copy(ee_hbm.at[pl.ds(ee_base, CH)], ee0, se0).start()

            def pair_body(cp, _):
                for b in range(2):
                    ci = cp * 2 + b
                    pk_v, ee_v = pk_bufs[b], ee_bufs[b]
                    npk, nee = pk_bufs[1 - b], ee_bufs[1 - b]
                    pltpu.make_async_copy(
                        packed.at[pl.ds(ci * CH, CH)], pk_v, sp_sems[b]).wait()
                    pltpu.make_async_copy(
                        ee_hbm.at[pl.ds(ee_base + ci * CH, CH)], ee_v,
                        se_sems[b]).wait()

                    @pl.when(ci + 1 < NCH)
                    def _():
                        off2 = (ci + 1) * CH
                        pltpu.make_async_copy(
                            packed.at[pl.ds(off2, CH)], npk,
                            sp_sems[1 - b]).start()
                        pltpu.make_async_copy(
                            ee_hbm.at[pl.ds(ee_base + off2, CH)], nee,
                            se_sems[1 - b]).start()

                    @plsc.parallel_loop(0, CH // L, 1, unroll=8)
                    def ib(k):
                        pk = pk_v[pl.ds(k * L, L)]
                        s = pk >> 15
                        d = pk & 32767
                        ex = ee_v[pl.ds(k * L, L)]
                        for c in range(CPW):
                            g = plsc.load_gather(tbl, [s + (c * N_PAD)])
                            plsc.addupdate_scatter(acc, [d + (c * N_PAD)], g * ex)

                return 0

            lax.fori_loop(0, NCH // 2, pair_body, 0)
            for c in range(CPW):
                pltpu.sync_copy(acc.at[pl.ds(c * N_PAD, N_PAD)], outT.at[c0 + c])

        npass = (NCHUNK + NW - 1) // NW
        for p in range(npass):
            if (p + 1) * NW <= NCHUNK:
                do_chunk(wid + p * NW)
            else:
                @pl.when(wid < NCHUNK - p * NW)
                def _():
                    do_chunk(wid + p * NW)

    return stage2


_make_sc_stage1 = functools.lru_cache(maxsize=None)(_make_sc_stage1)
_make_sc_stage2 = functools.lru_cache(maxsize=None)(_make_sc_stage2)


def _head_proj_mat(a):
    H, D = a.shape
    return (jnp.eye(H, dtype=a.dtype)[:, :, None] * a[None, :, :]).reshape(H, H * D)


def kernel(edge_index, x, W1, al1, ar1, b1, W2, al2, ar2, b2):
    packed = _pack_edges(edge_index).reshape(E_TOTAL)

    xT = jnp.pad(x.T, ((0, 0), (0, N_PAD - N_NODES)))
    W1T = W1.T
    AL1T = _head_proj_mat(al1)
    AR1T = _head_proj_mat(ar1)
    feat1T, el1T, er1T = _tc_proj1(xT, W1T, AL1T, AR1T)

    ee1, denP1 = _make_sc_stage1(H=4, NJ=8, CH=4000)(packed, el1T, er1T)
    out1T = _make_sc_stage2(C=256, CPW=4, NCHUNK=64, CPH=16, CH=8000)(
        packed, feat1T, ee1)

    W2T = jnp.pad(W2.T, ((0, 1), (0, 0)))
    AL2T = jnp.pad(al2, ((0, 0), (0, 1)))
    AR2T = jnp.pad(ar2, ((0, 0), (0, 1)))
    b1col = b1[:, None]
    feat2T, el2T, er2T = _tc_mid(out1T, denP1, W2T, AL2T, AR2T, b1col)

    ee2, denP2 = _make_sc_stage1(H=1, NJ=32, CH=2000)(packed, el2T, er2T)
    out2T = _make_sc_stage2(C=48, CPW=3, NCHUNK=16, CPH=16, CH=8000)(
        packed, feat2T, ee2)

    b2col = jnp.pad(b2, (0, 1))[:, None]
    res = _tc_final(out2T, denP2, b2col)
    return res[:47, :N_NODES].T


# speedup vs baseline: 1.4483x; 1.0933x over previous
import functools
import jax
import jax.numpy as jnp
from jax import lax
from jax.experimental import pallas as pl
from jax.experimental.pallas import tpu as pltpu
from jax.experimental.pallas import tpu_sc as plsc

N_NODES = 10000
N_PAD = 10240
E_TOTAL = 320000
NC, NS = 2, 16
NW = NC * NS
L = 16


def _pack_body(e_ref, p_ref):
    p_ref[...] = e_ref[0:1, :] * 32768 + e_ref[1:2, :]


def _pack_edges(edge_index):
    E = edge_index.shape[1]
    blk = 2560
    return pl.pallas_call(
        _pack_body,
        grid=(E // blk,),
        in_specs=[pl.BlockSpec((2, blk), lambda i: (0, i))],
        out_specs=pl.BlockSpec((1, blk), lambda i: (0, i)),
        out_shape=jax.ShapeDtypeStruct((1, E), jnp.int32),
    )(edge_index)


def _proj1_body(xT_ref, wT_ref, alT_ref, arT_ref, fT_ref, elT_ref, erT_ref):
    fT = jnp.dot(wT_ref[...], xT_ref[...], preferred_element_type=jnp.float32)
    fT_ref[...] = fT
    elT_ref[...] = jnp.dot(alT_ref[...], fT, preferred_element_type=jnp.float32)
    erT_ref[...] = jnp.dot(arT_ref[...], fT, preferred_element_type=jnp.float32)


def _tc_proj1(xT, W1T, AL1T, AR1T):
    blk = 512
    g = N_PAD // blk
    return pl.pallas_call(
        _proj1_body,
        grid=(g,),
        in_specs=[
            pl.BlockSpec((128, blk), lambda i: (0, i)),
            pl.BlockSpec((256, 128), lambda i: (0, 0)),
            pl.BlockSpec((4, 256), lambda i: (0, 0)),
            pl.BlockSpec((4, 256), lambda i: (0, 0)),
        ],
        out_specs=[
            pl.BlockSpec((256, blk), lambda i: (0, i)),
            pl.BlockSpec((4, blk), lambda i: (0, i)),
            pl.BlockSpec((4, blk), lambda i: (0, i)),
        ],
        out_shape=[
            jax.ShapeDtypeStruct((256, N_PAD), jnp.float32),
            jax.ShapeDtypeStruct((4, N_PAD), jnp.float32),
            jax.ShapeDtypeStruct((4, N_PAD), jnp.float32),
        ],
    )(xT, W1T, AL1T, AR1T)


def _mid_body(o1T_ref, dP_ref, w2T_ref, al2T_ref, ar2T_ref, b1_ref,
              f2T_ref, el2T_ref, er2T_ref):
    den = jnp.sum(dP_ref[...].reshape(4, 8, o1T_ref.shape[1]), axis=1)
    x = o1T_ref[...].reshape(4, 64, o1T_ref.shape[1]) / (den[:, None, :] + 1e-16)
    x = x.reshape(256, o1T_ref.shape[1]) + b1_ref[...]
    h1T = jnp.where(x > 0, x, jnp.exp(x) - 1.0)
    f2T = jnp.dot(w2T_ref[...], h1T, preferred_element_type=jnp.float32)
    f2T_ref[...] = f2T
    el2T_ref[...] = jnp.dot(al2T_ref[...], f2T, preferred_element_type=jnp.float32)
    er2T_ref[...] = jnp.dot(ar2T_ref[...], f2T, preferred_element_type=jnp.float32)


def _tc_mid(out1T, denP1, W2T, AL2T, AR2T, b1col):
    blk = 512
    g = N_PAD // blk
    return pl.pallas_call(
        _mid_body,
        grid=(g,),
        in_specs=[
            pl.BlockSpec((256, blk), lambda i: (0, i)),
            pl.BlockSpec((32, blk), lambda i: (0, i)),
            pl.BlockSpec((48, 256), lambda i: (0, 0)),
            pl.BlockSpec((1, 48), lambda i: (0, 0)),
            pl.BlockSpec((1, 48), lambda i: (0, 0)),
            pl.BlockSpec((256, 1), lambda i: (0, 0)),
        ],
        out_specs=[
            pl.BlockSpec((48, blk), lambda i: (0, i)),
            pl.BlockSpec((1, blk), lambda i: (0, i)),
            pl.BlockSpec((1, blk), lambda i: (0, i)),
        ],
        out_shape=[
            jax.ShapeDtypeStruct((48, N_PAD), jnp.float32),
            jax.ShapeDtypeStruct((1, N_PAD), jnp.float32),
            jax.ShapeDtypeStruct((1, N_PAD), jnp.float32),
        ],
    )(out1T, denP1, W2T, AL2T, AR2T, b1col)


def _final_body(o2T_ref, dP_ref, b2_ref, out_ref):
    den = jnp.sum(dP_ref[...], axis=0, keepdims=True)
    out_ref[...] = o2T_ref[...] / (den + 1e-16) + b2_ref[...]


def _tc_final(out2T, denP2, b2col):
    blk = 512
    g = N_PAD // blk
    return pl.pallas_call(
        _final_body,
        grid=(g,),
        in_specs=[
            pl.BlockSpec((48, blk), lambda i: (0, i)),
            pl.BlockSpec((32, blk), lambda i: (0, i)),
            pl.BlockSpec((48, 1), lambda i: (0, 0)),
        ],
        out_specs=pl.BlockSpec((48, blk), lambda i: (0, i)),
        out_shape=jax.ShapeDtypeStruct((48, N_PAD), jnp.float32),
    )(out2T, denP2, b2col)


@functools.lru_cache(maxsize=None)
def _mesh():
    return plsc.VectorSubcoreMesh(
        core_axis_name="c", subcore_axis_name="s",
        num_cores=NC, num_subcores=NS)


def _worker_id():
    return lax.axis_index("s") * NC + lax.axis_index("c")


def _zero_vmem(ref, n):
    z = jnp.zeros((L,), jnp.float32)

    def zb(i, _):
        ref[pl.ds(i * L, L)] = z
        return 0

    lax.fori_loop(0, n // L, zb, 0)


def _make_sc_stage1(H, NJ, CH):
    EPW = E_TOTAL // NJ

    @functools.partial(
        pl.kernel,
        mesh=_mesh(),
        compiler_params=pltpu.CompilerParams(needs_layout_passes=False),
        out_type=[
            jax.ShapeDtypeStruct((H * E_TOTAL,), jnp.float32),
            jax.ShapeDtypeStruct((H * NJ, N_PAD), jnp.float32),
        ],
        scratch_types=[
            pltpu.VMEM((N_PAD,), jnp.float32),
            pltpu.VMEM((N_PAD,), jnp.float32),
            pltpu.VMEM((N_PAD,), jnp.float32),
            pltpu.VMEM((CH,), jnp.int32),
            pltpu.VMEM((CH,), jnp.float32),
        ],
    )
    def stage1(packed, elT, erT, ee_out, denP, el_v, er_v, den_v, pk_v, ee_v):
        wid = _worker_id()
        h = wid // NJ
        j = wid % NJ
        pltpu.sync_copy(elT.at[h], el_v)
        pltpu.sync_copy(erT.at[h], er_v)
        _zero_vmem(den_v, N_PAD)
        base = j * EPW

        def chunk_body(ci, _):
            off = base + ci * CH
            pltpu.sync_copy(packed.at[pl.ds(off, CH)], pk_v)

            @plsc.parallel_loop(0, CH // L, 1, unroll=8)
            def ib(k):
                pk = pk_v[pl.ds(k * L, L)]
                s = pk >> 15
                d = pk & 32767
                e = plsc.load_gather(el_v, [s]) + plsc.load_gather(er_v, [d])
                e = jnp.where(e > 0, e, 0.2 * e)
                ex = jnp.exp(e)
                ee_v[pl.ds(k * L, L)] = ex
                plsc.addupdate_scatter(den_v, [d], ex)
            pltpu.sync_copy(ee_v, ee_out.at[pl.ds(h * E_TOTAL + off, CH)])
            return 0

        lax.fori_loop(0, EPW // CH, chunk_body, 0)
        pltpu.sync_copy(den_v, denP.at[h * NJ + j])

    return stage1


def _make_sc_stage2(C, CPW, NCHUNK, CPH, CH):
    NCH = E_TOTAL // CH
    assert NCH % 2 == 0

    @functools.partial(
        pl.kernel,
        mesh=_mesh(),
        compiler_params=pltpu.CompilerParams(needs_layout_passes=False),
        out_type=jax.ShapeDtypeStruct((C, N_PAD), jnp.float32),
        scratch_types=[
            pltpu.VMEM((CPW * N_PAD,), jnp.float32),
            pltpu.VMEM((CPW * N_PAD,), jnp.float32),
            pltpu.VMEM((CH,), jnp.int32),
            pltpu.VMEM((CH,), jnp.int32),
            pltpu.VMEM((CH,), jnp.float32),
            pltpu.VMEM((CH,), jnp.float32),
            pltpu.SemaphoreType.DMA,
            pltpu.SemaphoreType.DMA,
            pltpu.SemaphoreType.DMA,
            pltpu.SemaphoreType.DMA,
        ],
    )
    def stage2(packed, featT, ee_hbm, outT, tbl, acc,
               pk0, pk1, ee0, ee1, sp0, sp1, se0, se1):
        wid = _worker_id()
        pk_bufs = (pk0, pk1)
        ee_bufs = (ee0, ee1)
        sp_sems = (sp0, sp1)
        se_sems = (se0, se1)

        def do_chunk(chunk):
            c0 = chunk * CPW
            h = chunk // CPH
            ee_base = h * E_TOTAL
            for c in range(CPW):
                pltpu.sync_copy(featT.at[c0 + c], tbl.at[pl.ds(c * N_PAD, N_PAD)])
            _zero_vmem(acc, CPW * N_PAD)

            pltpu.make_async_copy(packed.at[pl.ds(0, CH)], pk0, sp0).start()
            pltpu.make_async_copy(ee_hbm.at[pl.ds(ee_base, CH)], ee0, se0).start()

            def pair_body(cp, _):
                for b in range(2):
                    ci = cp * 2 + b
                    pk_v, ee_v = pk_bufs[b], ee_bufs[b]
                    npk, nee = pk_bufs[1 - b], ee_bufs[1 - b]
                    pltpu.make_async_copy(
                        packed.at[pl.ds(ci * CH, CH)], pk_v, sp_sems[b]).wait()
                    pltpu.make_async_copy(
                        ee_hbm.at[pl.ds(ee_base + ci * CH, CH)], ee_v,
                        se_sems[b]).wait()

                    @pl.when(ci + 1 < NCH)
                    def _():
                        off2 = (ci + 1) * CH
                        pltpu.make_async_copy(
                            packed.at[pl.ds(off2, CH)], npk,
                            sp_sems[1 - b]).start()
                        pltpu.make_async_copy(
                            ee_hbm.at[pl.ds(ee_base + off2, CH)], nee,
                            se_sems[1 - b]).start()

                    @plsc.parallel_loop(0, CH // L, 1, unroll=4)
                    def ib(k):
                        pk = pk_v[pl.ds(k * L, L)]
                        s = pk >> 15
                        d = pk & 32767
                        ex = ee_v[pl.ds(k * L, L)]
                        for c in range(CPW):
                            g = plsc.load_gather(tbl, [s + (c * N_PAD)])
                            plsc.addupdate_scatter(acc, [d + (c * N_PAD)], g * ex)

                return 0

            lax.fori_loop(0, NCH // 2, pair_body, 0)
            for c in range(CPW):
                pltpu.sync_copy(acc.at[pl.ds(c * N_PAD, N_PAD)], outT.at[c0 + c])

        npass = (NCHUNK + NW - 1) // NW
        for p in range(npass):
            if (p + 1) * NW <= NCHUNK:
                do_chunk(wid + p * NW)
            else:
                @pl.when(wid < NCHUNK - p * NW)
                def _():
                    do_chunk(wid + p * NW)

    return stage2


_make_sc_stage1 = functools.lru_cache(maxsize=None)(_make_sc_stage1)
_make_sc_stage2 = functools.lru_cache(maxsize=None)(_make_sc_stage2)


def _head_proj_mat(a):
    H, D = a.shape
    return (jnp.eye(H, dtype=a.dtype)[:, :, None] * a[None, :, :]).reshape(H, H * D)


def kernel(edge_index, x, W1, al1, ar1, b1, W2, al2, ar2, b2):
    packed = _pack_edges(edge_index).reshape(E_TOTAL)

    xT = jnp.pad(x.T, ((0, 0), (0, N_PAD - N_NODES)))
    W1T = W1.T
    AL1T = _head_proj_mat(al1)
    AR1T = _head_proj_mat(ar1)
    feat1T, el1T, er1T = _tc_proj1(xT, W1T, AL1T, AR1T)

    ee1, denP1 = _make_sc_stage1(H=4, NJ=8, CH=4000)(packed, el1T, er1T)
    out1T = _make_sc_stage2(C=256, CPW=4, NCHUNK=64, CPH=16, CH=8000)(
        packed, feat1T, ee1)

    W2T = jnp.pad(W2.T, ((0, 1), (0, 0)))
    AL2T = jnp.pad(al2, ((0, 0), (0, 1)))
    AR2T = jnp.pad(ar2, ((0, 0), (0, 1)))
    b1col = b1[:, None]
    feat2T, el2T, er2T = _tc_mid(out1T, denP1, W2T, AL2T, AR2T, b1col)

    ee2, denP2 = _make_sc_stage1(H=1, NJ=32, CH=2000)(packed, el2T, er2T)
    out2T = _make_sc_stage2(C=48, CPW=2, NCHUNK=24, CPH=24, CH=8000)(
        packed, feat2T, ee2)

    b2col = jnp.pad(b2, (0, 1))[:, None]
    res = _tc_final(out2T, denP2, b2col)
    return res[:47, :N_NODES].T


# speedup vs baseline: 1.6005x; 1.1051x over previous
import functools
import jax
import jax.numpy as jnp
from jax import lax
from jax.experimental import pallas as pl
from jax.experimental.pallas import tpu as pltpu
from jax.experimental.pallas import tpu_sc as plsc

N_NODES = 10000
N_PAD = 10240
E_TOTAL = 320000
NC, NS = 2, 16
NW = NC * NS
L = 16


def _pack_body(e_ref, p_ref):
    p_ref[...] = e_ref[0:1, :] * 32768 + e_ref[1:2, :]


def _pack_edges(edge_index):
    E = edge_index.shape[1]
    blk = 2560
    return pl.pallas_call(
        _pack_body,
        grid=(E // blk,),
        in_specs=[pl.BlockSpec((2, blk), lambda i: (0, i))],
        out_specs=pl.BlockSpec((1, blk), lambda i: (0, i)),
        out_shape=jax.ShapeDtypeStruct((1, E), jnp.int32),
    )(edge_index)


def _proj1_body(xT_ref, wT_ref, alT_ref, arT_ref, fT_ref, elT_ref, erT_ref):
    fT = jnp.dot(wT_ref[...], xT_ref[...], preferred_element_type=jnp.float32)
    fT_ref[...] = fT
    elT_ref[...] = jnp.dot(alT_ref[...], fT, preferred_element_type=jnp.float32)
    erT_ref[...] = jnp.dot(arT_ref[...], fT, preferred_element_type=jnp.float32)


def _tc_proj1(xT, W1T, AL1T, AR1T):
    blk = 512
    g = N_PAD // blk
    return pl.pallas_call(
        _proj1_body,
        grid=(g,),
        in_specs=[
            pl.BlockSpec((128, blk), lambda i: (0, i)),
            pl.BlockSpec((256, 128), lambda i: (0, 0)),
            pl.BlockSpec((4, 256), lambda i: (0, 0)),
            pl.BlockSpec((4, 256), lambda i: (0, 0)),
        ],
        out_specs=[
            pl.BlockSpec((256, blk), lambda i: (0, i)),
            pl.BlockSpec((4, blk), lambda i: (0, i)),
            pl.BlockSpec((4, blk), lambda i: (0, i)),
        ],
        out_shape=[
            jax.ShapeDtypeStruct((256, N_PAD), jnp.float32),
            jax.ShapeDtypeStruct((4, N_PAD), jnp.float32),
            jax.ShapeDtypeStruct((4, N_PAD), jnp.float32),
        ],
    )(xT, W1T, AL1T, AR1T)


def _mid_body(o1T_ref, dP_ref, w2T_ref, al2T_ref, ar2T_ref, b1_ref,
              f2T_ref, el2T_ref, er2T_ref):
    den = jnp.sum(dP_ref[...].reshape(4, 8, o1T_ref.shape[1]), axis=1)
    x = o1T_ref[...].reshape(4, 64, o1T_ref.shape[1]) / (den[:, None, :] + 1e-16)
    x = x.reshape(256, o1T_ref.shape[1]) + b1_ref[...]
    h1T = jnp.where(x > 0, x, jnp.exp(x) - 1.0)
    f2T = jnp.dot(w2T_ref[...], h1T, preferred_element_type=jnp.float32)
    f2T_ref[...] = f2T
    el2T_ref[...] = jnp.dot(al2T_ref[...], f2T, preferred_element_type=jnp.float32)
    er2T_ref[...] = jnp.dot(ar2T_ref[...], f2T, preferred_element_type=jnp.float32)


def _tc_mid(out1T, denP1, W2T, AL2T, AR2T, b1col):
    blk = 512
    g = N_PAD // blk
    return pl.pallas_call(
        _mid_body,
        grid=(g,),
        in_specs=[
            pl.BlockSpec((256, blk), lambda i: (0, i)),
            pl.BlockSpec((32, blk), lambda i: (0, i)),
            pl.BlockSpec((48, 256), lambda i: (0, 0)),
            pl.BlockSpec((1, 48), lambda i: (0, 0)),
            pl.BlockSpec((1, 48), lambda i: (0, 0)),
            pl.BlockSpec((256, 1), lambda i: (0, 0)),
        ],
        out_specs=[
            pl.BlockSpec((48, blk), lambda i: (0, i)),
            pl.BlockSpec((1, blk), lambda i: (0, i)),
            pl.BlockSpec((1, blk), lambda i: (0, i)),
        ],
        out_shape=[
            jax.ShapeDtypeStruct((48, N_PAD), jnp.float32),
            jax.ShapeDtypeStruct((1, N_PAD), jnp.float32),
            jax.ShapeDtypeStruct((1, N_PAD), jnp.float32),
        ],
    )(out1T, denP1, W2T, AL2T, AR2T, b1col)


def _final_body(o2T_ref, dP_ref, b2_ref, out_ref):
    den = jnp.sum(dP_ref[...], axis=0, keepdims=True)
    out_ref[...] = o2T_ref[...] / (den + 1e-16) + b2_ref[...]


def _tc_final(out2T, denP2, b2col):
    blk = 512
    g = N_PAD // blk
    return pl.pallas_call(
        _final_body,
        grid=(g,),
        in_specs=[
            pl.BlockSpec((48, blk), lambda i: (0, i)),
            pl.BlockSpec((32, blk), lambda i: (0, i)),
            pl.BlockSpec((48, 1), lambda i: (0, 0)),
        ],
        out_specs=pl.BlockSpec((48, blk), lambda i: (0, i)),
        out_shape=jax.ShapeDtypeStruct((48, N_PAD), jnp.float32),
    )(out2T, denP2, b2col)


@functools.lru_cache(maxsize=None)
def _mesh():
    return plsc.VectorSubcoreMesh(
        core_axis_name="c", subcore_axis_name="s",
        num_cores=NC, num_subcores=NS)


def _worker_id():
    return lax.axis_index("s") * NC + lax.axis_index("c")


def _zero_vmem(ref, n):
    z = jnp.zeros((L,), jnp.float32)

    def zb(i, _):
        ref[pl.ds(i * L, L)] = z
        return 0

    lax.fori_loop(0, n // L, zb, 0)


def _make_sc_stage1(H, NJ, CH):
    EPW = E_TOTAL // NJ

    @functools.partial(
        pl.kernel,
        mesh=_mesh(),
        compiler_params=pltpu.CompilerParams(needs_layout_passes=False),
        out_type=[
            jax.ShapeDtypeStruct((H * E_TOTAL,), jnp.float32),
            jax.ShapeDtypeStruct((H * NJ, N_PAD), jnp.float32),
        ],
        scratch_types=[
            pltpu.VMEM((N_PAD,), jnp.float32),
            pltpu.VMEM((N_PAD,), jnp.float32),
            pltpu.VMEM((N_PAD,), jnp.float32),
            pltpu.VMEM((CH,), jnp.int32),
            pltpu.VMEM((CH,), jnp.float32),
        ],
    )
    def stage1(packed, elT, erT, ee_out, denP, el_v, er_v, den_v, pk_v, ee_v):
        wid = _worker_id()
        h = wid // NJ
        j = wid % NJ
        pltpu.sync_copy(elT.at[h], el_v)
        pltpu.sync_copy(erT.at[h], er_v)
        _zero_vmem(den_v, N_PAD)
        base = j * EPW

        def chunk_body(ci, _):
            off = base + ci * CH
            pltpu.sync_copy(packed.at[pl.ds(off, CH)], pk_v)

            @plsc.parallel_loop(0, CH // L, 1, unroll=8)
            def ib(k):
                pk = pk_v[pl.ds(k * L, L)]
                s = pk >> 15
                d = pk & 32767
                e = plsc.load_gather(el_v, [s]) + plsc.load_gather(er_v, [d])
                e = jnp.where(e > 0, e, 0.2 * e)
                ex = jnp.exp(e)
                ee_v[pl.ds(k * L, L)] = ex
                plsc.addupdate_scatter(den_v, [d], ex)
            pltpu.sync_copy(ee_v, ee_out.at[pl.ds(h * E_TOTAL + off, CH)])
            return 0

        lax.fori_loop(0, EPW // CH, chunk_body, 0)
        pltpu.sync_copy(den_v, denP.at[h * NJ + j])

    return stage1


def _make_sc_stage2(C, CPW, NCHUNK, CPH, CH):
    NCH = E_TOTAL // CH
    assert NCH % 2 == 0
    assert CPW % 2 == 0
    NPR = CPW // 2

    @functools.partial(
        pl.kernel,
        mesh=_mesh(),
        compiler_params=pltpu.CompilerParams(needs_layout_passes=False),
        out_type=jax.ShapeDtypeStruct((C, N_PAD), jnp.float32),
        scratch_types=[
            pltpu.VMEM((2 * N_PAD,), jnp.float32),
            pltpu.VMEM((NPR * N_PAD,), jnp.int32),
            pltpu.VMEM((CPW * N_PAD,), jnp.float32),
            pltpu.VMEM((CH,), jnp.int32),
            pltpu.VMEM((CH,), jnp.int32),
            pltpu.VMEM((CH,), jnp.float32),
            pltpu.VMEM((CH,), jnp.float32),
            pltpu.SemaphoreType.DMA,
            pltpu.SemaphoreType.DMA,
            pltpu.SemaphoreType.DMA,
            pltpu.SemaphoreType.DMA,
        ],
    )
    def stage2(packed, featT, ee_hbm, outT, stg, ptbl, acc,
               pk0, pk1, ee0, ee1, sp0, sp1, se0, se1):
        wid = _worker_id()
        pk_bufs = (pk0, pk1)
        ee_bufs = (ee0, ee1)
        sp_sems = (sp0, sp1)
        se_sems = (se0, se1)

        def do_chunk(chunk):
            c0 = chunk * CPW
            h = chunk // CPH
            ee_base = h * E_TOTAL
            for pr in range(NPR):
                pltpu.sync_copy(featT.at[c0 + 2 * pr],
                                stg.at[pl.ds(0, N_PAD)])
                pltpu.sync_copy(featT.at[c0 + 2 * pr + 1],
                                stg.at[pl.ds(N_PAD, N_PAD)])

                @plsc.parallel_loop(0, N_PAD // L, 1, unroll=4)
                def pack_body(n):
                    a = stg[pl.ds(n * L, L)]
                    b = stg[pl.ds(N_PAD + n * L, L)]
                    w = plsc.bitcast(
                        plsc.pack(a, b, format=plsc.PackFormat.INTERLEAVED),
                        jnp.int32)
                    ptbl[pl.ds(pr * N_PAD + n * L, L)] = w

            _zero_vmem(acc, CPW * N_PAD)

            pltpu.make_async_copy(packed.at[pl.ds(0, CH)], pk0, sp0).start()
            pltpu.make_async_copy(ee_hbm.at[pl.ds(ee_base, CH)], ee0, se0).start()

            def pair_body(cp, _):
                for b in range(2):
                    ci = cp * 2 + b
                    pk_v, ee_v = pk_bufs[b], ee_bufs[b]
                    npk, nee = pk_bufs[1 - b], ee_bufs[1 - b]
                    pltpu.make_async_copy(
                        packed.at[pl.ds(ci * CH, CH)], pk_v, sp_sems[b]).wait()
                    pltpu.make_async_copy(
                        ee_hbm.at[pl.ds(ee_base + ci * CH, CH)], ee_v,
                        se_sems[b]).wait()

                    @pl.when(ci + 1 < NCH)
                    def _():
                        off2 = (ci + 1) * CH
                        pltpu.make_async_copy(
                            packed.at[pl.ds(off2, CH)], npk,
                            sp_sems[1 - b]).start()
                        pltpu.make_async_copy(
                            ee_hbm.at[pl.ds(ee_base + off2, CH)], nee,
                            se_sems[1 - b]).start()

                    @plsc.parallel_loop(0, CH // L, 1, unroll=4)
                    def ib(k):
                        pk = pk_v[pl.ds(k * L, L)]
                        s = pk >> 15
                        d = pk & 32767
                        ex = ee_v[pl.ds(k * L, L)]
                        for pr in range(NPR):
                            g = plsc.load_gather(ptbl, [s + (pr * N_PAD)])
                            a, b = plsc.unpack(
                                plsc.bitcast(g, jnp.bfloat16),
                                format=plsc.PackFormat.INTERLEAVED,
                                preferred_element_type=jnp.float32)
                            plsc.addupdate_scatter(
                                acc, [d + (2 * pr) * N_PAD], a * ex)
                            plsc.addupdate_scatter(
                                acc, [d + (2 * pr + 1) * N_PAD], b * ex)

                return 0

            lax.fori_loop(0, NCH // 2, pair_body, 0)
            for c in range(CPW):
                pltpu.sync_copy(acc.at[pl.ds(c * N_PAD, N_PAD)], outT.at[c0 + c])

        npass = (NCHUNK + NW - 1) // NW
        for p in range(npass):
            if (p + 1) * NW <= NCHUNK:
                do_chunk(wid + p * NW)
            else:
                @pl.when(wid < NCHUNK - p * NW)
                def _():
                    do_chunk(wid + p * NW)

    return stage2


_make_sc_stage1 = functools.lru_cache(maxsize=None)(_make_sc_stage1)
_make_sc_stage2 = functools.lru_cache(maxsize=None)(_make_sc_stage2)


def _head_proj_mat(a):
    H, D = a.shape
    return (jnp.eye(H, dtype=a.dtype)[:, :, None] * a[None, :, :]).reshape(H, H * D)


def kernel(edge_index, x, W1, al1, ar1, b1, W2, al2, ar2, b2):
    packed = _pack_edges(edge_index).reshape(E_TOTAL)

    xT = jnp.pad(x.T, ((0, 0), (0, N_PAD - N_NODES)))
    W1T = W1.T
    AL1T = _head_proj_mat(al1)
    AR1T = _head_proj_mat(ar1)
    feat1T, el1T, er1T = _tc_proj1(xT, W1T, AL1T, AR1T)

    ee1, denP1 = _make_sc_stage1(H=4, NJ=8, CH=4000)(packed, el1T, er1T)
    out1T = _make_sc_stage2(C=256, CPW=4, NCHUNK=64, CPH=16, CH=8000)(
        packed, feat1T, ee1)

    W2T = jnp.pad(W2.T, ((0, 1), (0, 0)))
    AL2T = jnp.pad(al2, ((0, 0), (0, 1)))
    AR2T = jnp.pad(ar2, ((0, 0), (0, 1)))
    b1col = b1[:, None]
    feat2T, el2T, er2T = _tc_mid(out1T, denP1, W2T, AL2T, AR2T, b1col)

    ee2, denP2 = _make_sc_stage1(H=1, NJ=32, CH=2000)(packed, el2T, er2T)
    out2T = _make_sc_stage2(C=48, CPW=2, NCHUNK=24, CPH=24, CH=8000)(
        packed, feat2T, ee2)

    b2col = jnp.pad(b2, (0, 1))[:, None]
    res = _tc_final(out2T, denP2, b2col)
    return res[:47, :N_NODES].T


# speedup vs baseline: 1.6038x; 1.0021x over previous
import functools
import jax
import jax.numpy as jnp
from jax import lax
from jax.experimental import pallas as pl
from jax.experimental.pallas import tpu as pltpu
from jax.experimental.pallas import tpu_sc as plsc

N_NODES = 10000
N_PAD = 10240
E_TOTAL = 320000
NC, NS = 2, 16
NW = NC * NS
L = 16


def _pack_body(e_ref, p_ref):
    p_ref[...] = e_ref[0:1, :] * 32768 + e_ref[1:2, :]


def _pack_edges(edge_index):
    E = edge_index.shape[1]
    blk = 2560
    return pl.pallas_call(
        _pack_body,
        grid=(E // blk,),
        in_specs=[pl.BlockSpec((2, blk), lambda i: (0, i))],
        out_specs=pl.BlockSpec((1, blk), lambda i: (0, i)),
        out_shape=jax.ShapeDtypeStruct((1, E), jnp.int32),
    )(edge_index)


def _proj1_body(xT_ref, wT_ref, alT_ref, arT_ref, fT_ref, elT_ref, erT_ref):
    fT = jnp.dot(wT_ref[...], xT_ref[...], preferred_element_type=jnp.float32)
    fT_ref[...] = fT
    elT_ref[...] = jnp.dot(alT_ref[...], fT, preferred_element_type=jnp.float32)
    erT_ref[...] = jnp.dot(arT_ref[...], fT, preferred_element_type=jnp.float32)


def _tc_proj1(xT, W1T, AL1T, AR1T):
    blk = 512
    g = N_PAD // blk
    return pl.pallas_call(
        _proj1_body,
        grid=(g,),
        in_specs=[
            pl.BlockSpec((128, blk), lambda i: (0, i)),
            pl.BlockSpec((256, 128), lambda i: (0, 0)),
            pl.BlockSpec((4, 256), lambda i: (0, 0)),
            pl.BlockSpec((4, 256), lambda i: (0, 0)),
        ],
        out_specs=[
            pl.BlockSpec((256, blk), lambda i: (0, i)),
            pl.BlockSpec((4, blk), lambda i: (0, i)),
            pl.BlockSpec((4, blk), lambda i: (0, i)),
        ],
        out_shape=[
            jax.ShapeDtypeStruct((256, N_PAD), jnp.float32),
            jax.ShapeDtypeStruct((4, N_PAD), jnp.float32),
            jax.ShapeDtypeStruct((4, N_PAD), jnp.float32),
        ],
    )(xT, W1T, AL1T, AR1T)


def _mid_body(o1T_ref, dP_ref, w2T_ref, al2T_ref, ar2T_ref, b1_ref,
              f2T_ref, el2T_ref, er2T_ref):
    den = jnp.sum(dP_ref[...].reshape(4, 8, o1T_ref.shape[1]), axis=1)
    x = o1T_ref[...].reshape(4, 64, o1T_ref.shape[1]) / (den[:, None, :] + 1e-16)
    x = x.reshape(256, o1T_ref.shape[1]) + b1_ref[...]
    h1T = jnp.where(x > 0, x, jnp.exp(x) - 1.0)
    f2T = jnp.dot(w2T_ref[...], h1T, preferred_element_type=jnp.float32)
    f2T_ref[...] = f2T
    el2T_ref[...] = jnp.dot(al2T_ref[...], f2T, preferred_element_type=jnp.float32)
    er2T_ref[...] = jnp.dot(ar2T_ref[...], f2T, preferred_element_type=jnp.float32)


def _tc_mid(out1T, denP1, W2T, AL2T, AR2T, b1col):
    blk = 512
    g = N_PAD // blk
    return pl.pallas_call(
        _mid_body,
        grid=(g,),
        in_specs=[
            pl.BlockSpec((256, blk), lambda i: (0, i)),
            pl.BlockSpec((32, blk), lambda i: (0, i)),
            pl.BlockSpec((48, 256), lambda i: (0, 0)),
            pl.BlockSpec((1, 48), lambda i: (0, 0)),
            pl.BlockSpec((1, 48), lambda i: (0, 0)),
            pl.BlockSpec((256, 1), lambda i: (0, 0)),
        ],
        out_specs=[
            pl.BlockSpec((48, blk), lambda i: (0, i)),
            pl.BlockSpec((1, blk), lambda i: (0, i)),
            pl.BlockSpec((1, blk), lambda i: (0, i)),
        ],
        out_shape=[
            jax.ShapeDtypeStruct((48, N_PAD), jnp.float32),
            jax.ShapeDtypeStruct((1, N_PAD), jnp.float32),
            jax.ShapeDtypeStruct((1, N_PAD), jnp.float32),
        ],
    )(out1T, denP1, W2T, AL2T, AR2T, b1col)


def _final_body(o2T_ref, dP_ref, b2_ref, out_ref):
    den = jnp.sum(dP_ref[...], axis=0, keepdims=True)
    out_ref[...] = o2T_ref[...] / (den + 1e-16) + b2_ref[...]


def _tc_final(out2T, denP2, b2col):
    blk = 512
    g = N_PAD // blk
    return pl.pallas_call(
        _final_body,
        grid=(g,),
        in_specs=[
            pl.BlockSpec((48, blk), lambda i: (0, i)),
            pl.BlockSpec((32, blk), lambda i: (0, i)),
            pl.BlockSpec((48, 1), lambda i: (0, 0)),
        ],
        out_specs=pl.BlockSpec((48, blk), lambda i: (0, i)),
        out_shape=jax.ShapeDtypeStruct((48, N_PAD), jnp.float32),
    )(out2T, denP2, b2col)


@functools.lru_cache(maxsize=None)
def _mesh():
    return plsc.VectorSubcoreMesh(
        core_axis_name="c", subcore_axis_name="s",
        num_cores=NC, num_subcores=NS)


def _worker_id():
    return lax.axis_index("s") * NC + lax.axis_index("c")


def _zero_vmem(ref, n):
    z = jnp.zeros((L,), jnp.float32)

    def zb(i, _):
        ref[pl.ds(i * L, L)] = z
        return 0

    lax.fori_loop(0, n // L, zb, 0)


def _make_sc_stage1(H, NJ, CH):
    EPW = E_TOTAL // NJ

    @functools.partial(
        pl.kernel,
        mesh=_mesh(),
        compiler_params=pltpu.CompilerParams(needs_layout_passes=False),
        out_type=[
            jax.ShapeDtypeStruct((H * E_TOTAL,), jnp.float32),
            jax.ShapeDtypeStruct((H * NJ, N_PAD), jnp.float32),
        ],
        scratch_types=[
            pltpu.VMEM((N_PAD,), jnp.float32),
            pltpu.VMEM((N_PAD,), jnp.float32),
            pltpu.VMEM((N_PAD,), jnp.float32),
            pltpu.VMEM((CH,), jnp.int32),
            pltpu.VMEM((CH,), jnp.float32),
        ],
    )
    def stage1(packed, elT, erT, ee_out, denP, el_v, er_v, den_v, pk_v, ee_v):
        wid = _worker_id()
        h = wid // NJ
        j = wid % NJ
        pltpu.sync_copy(elT.at[h], el_v)
        pltpu.sync_copy(erT.at[h], er_v)
        _zero_vmem(den_v, N_PAD)
        base = j * EPW

        def chunk_body(ci, _):
            off = base + ci * CH
            pltpu.sync_copy(packed.at[pl.ds(off, CH)], pk_v)

            @plsc.parallel_loop(0, CH // L, 1, unroll=4)
            def ib(k):
                pk = pk_v[pl.ds(k * L, L)]
                s = pk >> 15
                d = pk & 32767
                e = plsc.load_gather(el_v, [s]) + plsc.load_gather(er_v, [d])
                e = jnp.where(e > 0, e, 0.2 * e)
                ex = jnp.exp(e)
                ee_v[pl.ds(k * L, L)] = ex
                plsc.addupdate_scatter(den_v, [d], ex)
            pltpu.sync_copy(ee_v, ee_out.at[pl.ds(h * E_TOTAL + off, CH)])
            return 0

        lax.fori_loop(0, EPW // CH, chunk_body, 0)
        pltpu.sync_copy(den_v, denP.at[h * NJ + j])

    return stage1


def _make_sc_stage2(C, CPW, NCHUNK, CPH, CH):
    NCH = E_TOTAL // CH
    assert NCH % 2 == 0
    assert CPW % 2 == 0
    NPR = CPW // 2

    @functools.partial(
        pl.kernel,
        mesh=_mesh(),
        compiler_params=pltpu.CompilerParams(needs_layout_passes=False),
        out_type=jax.ShapeDtypeStruct((C, N_PAD), jnp.float32),
        scratch_types=[
            pltpu.VMEM((2 * N_PAD,), jnp.float32),
            pltpu.VMEM((NPR * N_PAD,), jnp.int32),
            pltpu.VMEM((CPW * N_PAD,), jnp.float32),
            pltpu.VMEM((CH,), jnp.int32),
            pltpu.VMEM((CH,), jnp.int32),
            pltpu.VMEM((CH,), jnp.float32),
            pltpu.VMEM((CH,), jnp.float32),
            pltpu.SemaphoreType.DMA,
            pltpu.SemaphoreType.DMA,
            pltpu.SemaphoreType.DMA,
            pltpu.SemaphoreType.DMA,
        ],
    )
    def stage2(packed, featT, ee_hbm, outT, stg, ptbl, acc,
               pk0, pk1, ee0, ee1, sp0, sp1, se0, se1):
        wid = _worker_id()
        pk_bufs = (pk0, pk1)
        ee_bufs = (ee0, ee1)
        sp_sems = (sp0, sp1)
        se_sems = (se0, se1)

        def do_chunk(chunk):
            c0 = chunk * CPW
            h = chunk // CPH
            ee_base = h * E_TOTAL
            for pr in range(NPR):
                pltpu.sync_copy(featT.at[c0 + 2 * pr],
                                stg.at[pl.ds(0, N_PAD)])
                pltpu.sync_copy(featT.at[c0 + 2 * pr + 1],
                                stg.at[pl.ds(N_PAD, N_PAD)])

                @plsc.parallel_loop(0, N_PAD // L, 1, unroll=4)
                def pack_body(n):
                    a = stg[pl.ds(n * L, L)]
                    b = stg[pl.ds(N_PAD + n * L, L)]
                    w = plsc.bitcast(
                        plsc.pack(a, b, format=plsc.PackFormat.INTERLEAVED),
                        jnp.int32)
                    ptbl[pl.ds(pr * N_PAD + n * L, L)] = w

            _zero_vmem(acc, CPW * N_PAD)

            pltpu.make_async_copy(packed.at[pl.ds(0, CH)], pk0, sp0).start()
            pltpu.make_async_copy(ee_hbm.at[pl.ds(ee_base, CH)], ee0, se0).start()

            def pair_body(cp, _):
                for b in range(2):
                    ci = cp * 2 + b
                    pk_v, ee_v = pk_bufs[b], ee_bufs[b]
                    npk, nee = pk_bufs[1 - b], ee_bufs[1 - b]
                    pltpu.make_async_copy(
                        packed.at[pl.ds(ci * CH, CH)], pk_v, sp_sems[b]).wait()
                    pltpu.make_async_copy(
                        ee_hbm.at[pl.ds(ee_base + ci * CH, CH)], ee_v,
                        se_sems[b]).wait()

                    @pl.when(ci + 1 < NCH)
                    def _():
                        off2 = (ci + 1) * CH
                        pltpu.make_async_copy(
                            packed.at[pl.ds(off2, CH)], npk,
                            sp_sems[1 - b]).start()
                        pltpu.make_async_copy(
                            ee_hbm.at[pl.ds(ee_base + off2, CH)], nee,
                            se_sems[1 - b]).start()

                    @plsc.parallel_loop(0, CH // L, 1, unroll=4)
                    def ib(k):
                        pk = pk_v[pl.ds(k * L, L)]
                        s = pk >> 15
                        d = pk & 32767
                        ex = ee_v[pl.ds(k * L, L)]
                        for pr in range(NPR):
                            g = plsc.load_gather(ptbl, [s + (pr * N_PAD)])
                            a, b = plsc.unpack(
                                plsc.bitcast(g, jnp.bfloat16),
                                format=plsc.PackFormat.INTERLEAVED,
                                preferred_element_type=jnp.float32)
                            plsc.addupdate_scatter(
                                acc, [d + (2 * pr) * N_PAD], a * ex)
                            plsc.addupdate_scatter(
                                acc, [d + (2 * pr + 1) * N_PAD], b * ex)

                return 0

            lax.fori_loop(0, NCH // 2, pair_body, 0)
            for c in range(CPW):
                pltpu.sync_copy(acc.at[pl.ds(c * N_PAD, N_PAD)], outT.at[c0 + c])

        npass = (NCHUNK + NW - 1) // NW
        for p in range(npass):
            if (p + 1) * NW <= NCHUNK:
                do_chunk(wid + p * NW)
            else:
                @pl.when(wid < NCHUNK - p * NW)
                def _():
                    do_chunk(wid + p * NW)

    return stage2


_make_sc_stage1 = functools.lru_cache(maxsize=None)(_make_sc_stage1)
_make_sc_stage2 = functools.lru_cache(maxsize=None)(_make_sc_stage2)


def _head_proj_mat(a):
    H, D = a.shape
    return (jnp.eye(H, dtype=a.dtype)[:, :, None] * a[None, :, :]).reshape(H, H * D)


def kernel(edge_index, x, W1, al1, ar1, b1, W2, al2, ar2, b2):
    packed = _pack_edges(edge_index).reshape(E_TOTAL)

    xT = jnp.pad(x.T, ((0, 0), (0, N_PAD - N_NODES)))
    W1T = W1.T
    AL1T = _head_proj_mat(al1)
    AR1T = _head_proj_mat(ar1)
    feat1T, el1T, er1T = _tc_proj1(xT, W1T, AL1T, AR1T)

    ee1, denP1 = _make_sc_stage1(H=4, NJ=8, CH=4000)(packed, el1T, er1T)
    out1T = _make_sc_stage2(C=256, CPW=4, NCHUNK=64, CPH=16, CH=8000)(
        packed, feat1T, ee1)

    W2T = jnp.pad(W2.T, ((0, 1), (0, 0)))
    AL2T = jnp.pad(al2, ((0, 0), (0, 1)))
    AR2T = jnp.pad(ar2, ((0, 0), (0, 1)))
    b1col = b1[:, None]
    feat2T, el2T, er2T = _tc_mid(out1T, denP1, W2T, AL2T, AR2T, b1col)

    ee2, denP2 = _make_sc_stage1(H=1, NJ=32, CH=2000)(packed, el2T, er2T)
    out2T = _make_sc_stage2(C=48, CPW=2, NCHUNK=24, CPH=24, CH=8000)(
        packed, feat2T, ee2)

    b2col = jnp.pad(b2, (0, 1))[:, None]
    res = _tc_final(out2T, denP2, b2col)
    return res[:47, :N_NODES].T
